# baseline (device time: 80742 ns/iter reference)
import jax
import jax.numpy as jnp
from jax import lax
from jax.experimental import pallas as pl
from jax.experimental.pallas import tpu as pltpu

N_DEV = 32
CAP = 48
N_CHUNK = 2
CH = N_DEV // N_CHUNK


def kernel(x, router_W, route_idx, expert_W):
    n_tok, d = x.shape
    e_per, _, h = expert_W.shape
    n_exp = N_DEV * e_per
    blk = e_per * CAP

    assert e_per == 2

    def body(x_ref, rw_ref, idx_ref, ew_ref, out_ref,
             send_ref, recv_ref, res_ref, ret_ref,
             p1_send, p1_recv, p2_send, p2_recv):
        my = lax.axis_index("i")

        barrier_sem = pltpu.get_barrier_semaphore()
        for off in range(1, N_DEV):
            peer = lax.rem(my + off, N_DEV)
            pl.semaphore_signal(
                barrier_sem, inc=1,
                device_id=(peer,), device_id_type=pl.DeviceIdType.MESH,
            )
        pl.semaphore_wait(barrier_sem, N_DEV - 1)

        xv32 = x_ref[...]
        xv = xv32.astype(jnp.bfloat16)
        scores = jnp.dot(xv32, rw_ref[...], preferred_element_type=jnp.float32)
        mx = jnp.max(scores, axis=-1, keepdims=True)
        p = jnp.exp(scores - mx)
        probs = p / jnp.sum(p, axis=-1, keepdims=True)
        e0c = idx_ref[:, 0:1]
        e1c = idx_ref[:, 1:2]
        eid = lax.broadcasted_iota(jnp.int32, (n_tok, n_exp), 1)
        one0 = eid == e0c
        one1 = eid == e1c
        g0 = jnp.sum(probs * one0.astype(jnp.float32), axis=-1, keepdims=True)
        g1 = jnp.sum(probs * one1.astype(jnp.float32), axis=-1, keepdims=True)
        gs = g0 + g1
        w0 = g0 / gs
        w1 = g1 / gs

        e0r = jnp.transpose(e0c)
        e1r = jnp.transpose(e1c)
        w0r = jnp.transpose(w0)
        w1r = jnp.transpose(w1)
        eidT = lax.broadcasted_iota(jnp.int32, (n_exp, n_tok), 0)
        mT0 = eidT == e0r
        mT1 = eidT == e1r
        mT = mT0 | mT1
        gateT = (jnp.where(mT0, w0r, 0.0)
                 + jnp.where(mT1, w1r, 0.0))
        ti = lax.broadcasted_iota(jnp.int32, (n_tok, n_tok), 0)
        tj = lax.broadcasted_iota(jnp.int32, (n_tok, n_tok), 1)
        up_tri = (ti < tj).astype(jnp.bfloat16)
        ranksT = jnp.dot(mT.astype(jnp.bfloat16), up_tri,
                         preferred_element_type=jnp.float32).astype(jnp.int32)

        c3 = lax.broadcasted_iota(jnp.int32, (n_exp, CAP, n_tok), 1)
        hit3 = (jnp.broadcast_to(ranksT[:, None, :], (n_exp, CAP, n_tok))
                == c3) & jnp.broadcast_to(mT[:, None, :], (n_exp, CAP, n_tok))
        s_all = hit3.astype(jnp.bfloat16).reshape(n_exp * CAP, n_tok)
        gate3 = jnp.broadcast_to(
            gateT[:, None, :], (n_exp, CAP, n_tok))
        gs_all = jnp.where(hit3, gate3, 0.0
                           ).astype(jnp.bfloat16).reshape(n_exp * CAP, n_tok)

        send_ref[...] = jnp.dot(
            s_all, xv, preferred_element_type=jnp.float32
        ).astype(jnp.bfloat16).reshape(N_DEV, e_per, CAP, d)

        for dd in range(N_DEV):
            dsc = pltpu.make_async_remote_copy(
                src_ref=send_ref.at[pl.ds(dd, 1)],
                dst_ref=recv_ref.at[pl.ds(my, 1)],
                send_sem=p1_send.at[dd],
                recv_sem=p1_recv.at[my],
                device_id=(dd,),
                device_id_type=pl.DeviceIdType.MESH,
            )
            pl.when(my != dd)(dsc.start)
        recv_ref[pl.ds(my, 1)] = send_ref[pl.ds(my, 1)]

        def recv_wait(recv_buf, recv_sems, p):
            dsc = pltpu.make_async_remote_copy(
                src_ref=recv_buf.at[pl.ds(0, 1)],
                dst_ref=recv_buf.at[pl.ds(p, 1)],
                send_sem=recv_sems.at[0],
                recv_sem=recv_sems.at[p],
                device_id=(0,),
                device_id_type=pl.DeviceIdType.MESH,
            )
            pl.when(my != p)(dsc.wait_recv)

        wb = ew_ref[...].astype(jnp.bfloat16)
        for c in range(N_CHUNK):
            lo = c * CH
            for src in range(lo, lo + CH):
                recv_wait(recv_ref, p1_recv, src)
            rv = recv_ref[pl.ds(lo, CH)]
            outs = []
            for k in range(e_per):
                xin = rv[:, k].reshape(CH * CAP, d)
                ok = jnp.dot(xin, wb[k], preferred_element_type=jnp.float32)
                outs.append(ok.astype(jnp.bfloat16).reshape(CH, CAP, h))
            res_ref[pl.ds(lo, CH)] = jnp.stack(outs, axis=1)
            for src in range(lo, lo + CH):
                dsc = pltpu.make_async_remote_copy(
                    src_ref=res_ref.at[pl.ds(src, 1)],
                    dst_ref=ret_ref.at[pl.ds(my, 1)],
                    send_sem=p2_send.at[src],
                    recv_sem=p2_recv.at[my],
                    device_id=(src,),
                    device_id_type=pl.DeviceIdType.MESH,
                )
                pl.when(my != src)(dsc.start)
        ret_ref[pl.ds(my, 1)] = res_ref[pl.ds(my, 1)]

        acc = jnp.zeros((n_tok, h), jnp.float32)
        for oc in range(N_CHUNK):
            lo = oc * CH
            for o in range(lo, lo + CH):
                recv_wait(ret_ref, p2_recv, o)
            rets = ret_ref[pl.ds(lo, CH)].reshape(CH * blk, h)
            acc = acc + lax.dot_general(
                gs_all[lo * blk:(lo + CH) * blk, :], rets,
                dimension_numbers=(((0,), (0,)), ((), ())),
                preferred_element_type=jnp.float32,
            )
        out_ref[...] = acc

        for dd in range(N_DEV):
            d1 = pltpu.make_async_remote_copy(
                src_ref=send_ref.at[pl.ds(dd, 1)],
                dst_ref=recv_ref.at[pl.ds(0, 1)],
                send_sem=p1_send.at[dd],
                recv_sem=p1_recv.at[0],
                device_id=(0,),
                device_id_type=pl.DeviceIdType.MESH,
            )
            pl.when(my != dd)(d1.wait_send)
            d2 = pltpu.make_async_remote_copy(
                src_ref=res_ref.at[pl.ds(dd, 1)],
                dst_ref=ret_ref.at[pl.ds(0, 1)],
                send_sem=p2_send.at[dd],
                recv_sem=p2_recv.at[0],
                device_id=(0,),
                device_id_type=pl.DeviceIdType.MESH,
            )
            pl.when(my != dd)(d2.wait_send)

    return pl.pallas_call(
        body,
        out_shape=jax.ShapeDtypeStruct((n_tok, h), jnp.float32),
        in_specs=[
            pl.BlockSpec(memory_space=pltpu.VMEM),
            pl.BlockSpec(memory_space=pltpu.VMEM),
            pl.BlockSpec(memory_space=pltpu.VMEM),
            pl.BlockSpec(memory_space=pltpu.VMEM),
        ],
        out_specs=pl.BlockSpec(memory_space=pltpu.VMEM),
        scratch_shapes=[
            pltpu.VMEM((N_DEV, e_per, CAP, d), jnp.bfloat16),
            pltpu.VMEM((N_DEV, e_per, CAP, d), jnp.bfloat16),
            pltpu.VMEM((N_DEV, e_per, CAP, h), jnp.bfloat16),
            pltpu.VMEM((N_DEV, e_per, CAP, h), jnp.bfloat16),
            pltpu.SemaphoreType.DMA((N_DEV,)),
            pltpu.SemaphoreType.DMA((N_DEV,)),
            pltpu.SemaphoreType.DMA((N_DEV,)),
            pltpu.SemaphoreType.DMA((N_DEV,)),
        ],
        compiler_params=pltpu.CompilerParams(
            collective_id=0,
            vmem_limit_bytes=100 * 1024 * 1024,
        ),
    )(x, router_W, route_idx, expert_W)
